# baseline (speedup 1.0000x reference)
import functools

import jax
import jax.numpy as jnp
from jax import lax
from jax.experimental import pallas as pl
from jax.experimental.pallas import tpu as pltpu
from jax.experimental.pallas import tpu_sc as plsc

_N = 100000
_D = 512
_NC = 2
_NS = 16
_NW = _NC * _NS
_R = 40
_CHUNKS = _N // _R
_KMAX = -(-_CHUNKS // _NW)
_G = _D // 16
_RU = 20

_BN = 2000


def _tc_u_block(mask_ref, x_ref, xt_ref, u_ref):
    m = mask_ref[:]
    u_ref[:] = jnp.where(m, xt_ref[:], x_ref[:])


def _tc_u(mask, x, x_tilde):
    n, d = x.shape
    return pl.pallas_call(
        _tc_u_block,
        grid=(n // _BN,),
        in_specs=[
            pl.BlockSpec((1, d), lambda i: (0, 0)),
            pl.BlockSpec((_BN, d), lambda i: (i, 0)),
            pl.BlockSpec((_BN, d), lambda i: (i, 0)),
        ],
        out_specs=pl.BlockSpec((_BN, d), lambda i: (i, 0)),
        out_shape=jax.ShapeDtypeStruct((n, d), x.dtype),
    )(mask, x, x_tilde)


def _make_sc_ut():
    mesh = plsc.VectorSubcoreMesh(core_axis_name="c", subcore_axis_name="s")

    @functools.partial(
        pl.kernel,
        mesh=mesh,
        out_type=jax.ShapeDtypeStruct((_N, _D), jnp.float32),
        scratch_types=[
            pltpu.VMEM((_D,), jnp.int32),
            pltpu.VMEM((_R, _D), jnp.float32),
            pltpu.VMEM((_R, _D), jnp.float32),
            pltpu.VMEM((_R, _D), jnp.float32),
            pltpu.VMEM((_R, _D), jnp.float32),
            pltpu.SemaphoreType.DMA,
            pltpu.SemaphoreType.DMA,
            pltpu.SemaphoreType.DMA,
            pltpu.SemaphoreType.DMA,
        ],
    )
    def sc_ut(mask_hbm, x_hbm, xt_hbm, ut_hbm,
              mask_v, x_v0, x_v1, xt_v0, xt_v1,
              in_sem0, in_sem1, out_sem0, out_sem1):
        x_v = (x_v0, x_v1)
        xt_v = (xt_v0, xt_v1)
        in_sem = (in_sem0, in_sem1)
        out_sem = (out_sem0, out_sem1)

        wid = lax.axis_index("s") * _NC + lax.axis_index("c")
        pltpu.sync_copy(mask_hbm, mask_v)

        def rows_of(k):
            return pl.ds((wid + k * _NW) * _R, _R)

        def start_in(k, b):
            pltpu.async_copy(x_hbm.at[rows_of(k)], x_v[b], in_sem[b])
            pltpu.async_copy(xt_hbm.at[rows_of(k)], xt_v[b], in_sem[b])

        def wait_in(k, b):
            pltpu.make_async_copy(x_hbm.at[rows_of(k)], x_v[b], in_sem[b]).wait()
            pltpu.make_async_copy(xt_hbm.at[rows_of(k)], xt_v[b], in_sem[b]).wait()

        def start_out(k, b):
            pltpu.async_copy(xt_v[b], ut_hbm.at[rows_of(k)], out_sem[b])

        def wait_out(k, b):
            pltpu.make_async_copy(xt_v[b], ut_hbm.at[rows_of(k)], out_sem[b]).wait()

        def valid(k):
            return wid + k * _NW < _CHUNKS

        def compute(b):
            for g in range(_G):
                mb = mask_v[pl.ds(g * 16, 16)] != 0

                def rows(i, c, mb=mb, g=g, b=b):
                    for j in range(_RU):
                        r = i * _RU + j
                        xv = x_v[b][r, pl.ds(g * 16, 16)]
                        tv = xt_v[b][r, pl.ds(g * 16, 16)]
                        xt_v[b][r, pl.ds(g * 16, 16)] = jnp.where(mb, xv, tv)
                    return c

                lax.fori_loop(0, _R // _RU, rows, 0)

        start_in(0, 0)

        def step(k2, carry):
            for b in range(2):
                k = k2 * 2 + b

                @pl.when(valid(k + 1))
                def _(k=k, b=b):
                    @pl.when(k >= 1)
                    def _(k=k, b=b):
                        wait_out(k - 1, 1 - b)

                    start_in(k + 1, 1 - b)

                @pl.when(valid(k))
                def _(k=k, b=b):
                    wait_in(k, b)
                    compute(b)
                    start_out(k, b)

            return carry

        lax.fori_loop(0, (_KMAX + 1) // 2, step, 0)

        for k in (_KMAX - 3, _KMAX - 2, _KMAX - 1):
            @pl.when(valid(k) & jnp.logical_not(valid(k + 2)))
            def _(k=k):
                wait_out(k, k % 2)

    return sc_ut


_sc_ut = _make_sc_ut()


@jax.jit
def kernel(x, x_tilde):
    n, d = x.shape
    bool_swap = jax.random.bernoulli(jax.random.key(42), 0.5, (d,))
    mask_i = bool_swap.astype(jnp.int32)
    ut = _sc_ut(mask_i, x, x_tilde)
    u = _tc_u(bool_swap[None, :], x, x_tilde)
    return (u, ut)


# speedup vs baseline: 1.5044x; 1.5044x over previous
import functools

import jax
import jax.numpy as jnp
from jax import lax
from jax.experimental import pallas as pl
from jax.experimental.pallas import tpu as pltpu
from jax.experimental.pallas import tpu_sc as plsc

_N = 100000
_D = 512
_NC = 2
_NS = 16
_NW = _NC * _NS
_R = 32
_CHUNKS = _N // _R
_KMAX = -(-_CHUNKS // _NW)
_NB = 3
_G = _D // 16
_RU = 16

_BN = 2000


def _tc_u_block(mask_ref, x_ref, xt_ref, u_ref):
    m = mask_ref[:]
    u_ref[:] = jnp.where(m, xt_ref[:], x_ref[:])


def _tc_u(mask, x, x_tilde):
    n, d = x.shape
    return pl.pallas_call(
        _tc_u_block,
        grid=(n // _BN,),
        in_specs=[
            pl.BlockSpec((1, d), lambda i: (0, 0)),
            pl.BlockSpec((_BN, d), lambda i: (i, 0)),
            pl.BlockSpec((_BN, d), lambda i: (i, 0)),
        ],
        out_specs=pl.BlockSpec((_BN, d), lambda i: (i, 0)),
        out_shape=jax.ShapeDtypeStruct((n, d), x.dtype),
    )(mask, x, x_tilde)


def _make_sc_ut():
    mesh = plsc.VectorSubcoreMesh(core_axis_name="c", subcore_axis_name="s")

    @functools.partial(
        pl.kernel,
        mesh=mesh,
        out_type=jax.ShapeDtypeStruct((_N, _D), jnp.float32),
        scratch_types=(
            [pltpu.VMEM((_D,), jnp.int32)]
            + [pltpu.VMEM((_R, _D), jnp.float32) for _ in range(2 * _NB)]
            + [pltpu.SemaphoreType.DMA for _ in range(2 * _NB)]
        ),
    )
    def sc_ut(mask_hbm, x_hbm, xt_hbm, ut_hbm,
              mask_v, x_v0, x_v1, x_v2, xt_v0, xt_v1, xt_v2,
              in_sem0, in_sem1, in_sem2, out_sem0, out_sem1, out_sem2):
        x_v = (x_v0, x_v1, x_v2)
        xt_v = (xt_v0, xt_v1, xt_v2)
        in_sem = (in_sem0, in_sem1, in_sem2)
        out_sem = (out_sem0, out_sem1, out_sem2)

        wid = lax.axis_index("s") * _NC + lax.axis_index("c")
        pltpu.sync_copy(mask_hbm, mask_v)

        def rows_of(k):
            return pl.ds((wid + k * _NW) * _R, _R)

        def start_in(k, b):
            pltpu.async_copy(x_hbm.at[rows_of(k)], x_v[b], in_sem[b])
            pltpu.async_copy(xt_hbm.at[rows_of(k)], xt_v[b], in_sem[b])

        def wait_in(k, b):
            pltpu.make_async_copy(x_hbm.at[rows_of(k)], x_v[b], in_sem[b]).wait()
            pltpu.make_async_copy(xt_hbm.at[rows_of(k)], xt_v[b], in_sem[b]).wait()

        def start_out(k, b):
            pltpu.async_copy(xt_v[b], ut_hbm.at[rows_of(k)], out_sem[b])

        def wait_out(k, b):
            pltpu.make_async_copy(xt_v[b], ut_hbm.at[rows_of(k)], out_sem[b]).wait()

        def valid(k):
            return wid + k * _NW < _CHUNKS

        def compute(b):
            def grp(g, c, b=b):
                off = g * 16
                mb = mask_v[pl.ds(off, 16)] != 0
                for r in range(_R):
                    xv = x_v[b][r, pl.ds(off, 16)]
                    tv = xt_v[b][r, pl.ds(off, 16)]
                    xt_v[b][r, pl.ds(off, 16)] = jnp.where(mb, xv, tv)
                return c

            lax.fori_loop(0, _G, grp, 0)

        start_in(0, 0)

        def step(k3, carry):
            for b in range(_NB):
                k = k3 * _NB + b

                nb = (b + 1) % _NB

                @pl.when(valid(k + 1))
                def _(k=k, nb=nb):
                    @pl.when(k >= 2)
                    def _(k=k, nb=nb):
                        wait_out(k - 2, nb)

                    start_in(k + 1, nb)

                @pl.when(valid(k))
                def _(k=k, b=b):
                    wait_in(k, b)
                    compute(b)
                    start_out(k, b)

            return carry

        lax.fori_loop(0, -(-(_KMAX + 1) // _NB), step, 0)

        for k in range(_KMAX - 4, _KMAX):
            @pl.when(valid(k) & jnp.logical_not(valid(k + _NB)))
            def _(k=k):
                wait_out(k, k % _NB)

    return sc_ut


_sc_ut = _make_sc_ut()


@jax.jit
def kernel(x, x_tilde):
    n, d = x.shape
    bool_swap = jax.random.bernoulli(jax.random.key(42), 0.5, (d,))
    mask_i = bool_swap.astype(jnp.int32)
    ut = _sc_ut(mask_i, x, x_tilde)
    u = _tc_u(bool_swap[None, :], x, x_tilde)
    return (u, ut)
